# baseline (speedup 1.0000x reference)
import functools

import numpy as np
import jax
import jax.numpy as jnp
from jax import lax
from jax.experimental import pallas as pl
from jax.experimental.pallas import tpu as pltpu
from jax.experimental.pallas import tpu_sc as plsc

K_NB = 16
NGR = 4
EPS = 1e-5
SLOPE = 0.2
NW = 32


def _knn_body(cq_ref, ck_ref, out_ref, *, M):
    b = pl.program_id(0)
    q = cq_ref[0]
    k = ck_ref[0]
    sq_q = q[:, 0:1] * q[:, 0:1] + q[:, 1:2] * q[:, 1:2] + q[:, 2:3] * q[:, 2:3]
    sq_k = k[0:1] * k[0:1] + k[1:2] * k[1:2] + k[2:3] * k[2:3]
    dot = lax.dot_general(q.astype(jnp.bfloat16), k.astype(jnp.bfloat16),
                          (((1,), (0,)), ((), ())),
                          preferred_element_type=jnp.float32)
    d = (sq_q + sq_k) - 2.0 * dot
    iota = lax.broadcasted_iota(jnp.int32, d.shape, 1)
    cols = []
    for _ in range(K_NB):
        m = jnp.min(d, axis=1, keepdims=True)
        eq = d == m
        idxj = jnp.min(jnp.where(eq, iota, M), axis=1, keepdims=True)
        cols.append(idxj)
        d = jnp.where(iota == idxj, jnp.float32(jnp.inf), d)
    out_ref[0] = jnp.concatenate(cols, axis=1) + b * M


def _knn(cqT, ck):
    B, N, _ = cqT.shape
    M = ck.shape[2]
    NT = 1024
    return pl.pallas_call(
        functools.partial(_knn_body, M=M),
        grid=(B, N // NT),
        in_specs=[pl.BlockSpec((1, NT, 3), lambda b, t: (b, t, 0)),
                  pl.BlockSpec((1, 3, M), lambda b, t: (b, 0, 0))],
        out_specs=pl.BlockSpec((1, NT, K_NB), lambda b, t: (b, t, 0)),
        out_shape=jax.ShapeDtypeStruct((B, N, K_NB), jnp.int32),
    )(cqT, ck)


def _gemm_body(a_ref, w_ref, o_ref):
    o_ref[0] = lax.dot_general(a_ref[0], w_ref[...], (((1,), (0,)), ((), ())),
                               preferred_element_type=jnp.float32)


def _gemm(a, w):
    B, R, Kd = a.shape
    Co = w.shape[1]
    RT = min(R, 512)
    return pl.pallas_call(
        _gemm_body,
        grid=(B, R // RT),
        in_specs=[pl.BlockSpec((1, RT, Kd), lambda b, t: (b, t, 0)),
                  pl.BlockSpec((Kd, Co), lambda b, t: (0, 0))],
        out_specs=pl.BlockSpec((1, RT, Co), lambda b, t: (b, t, 0)),
        out_shape=jax.ShapeDtypeStruct((B, R, Co), jnp.float32),
    )(a, w)


def _make_sc_gather(TOT, C, Q=4):
    R = TOT // NW
    CH = C // 16
    CHG = CH // NGR
    mesh = plsc.VectorSubcoreMesh(core_axis_name="c", subcore_axis_name="s")

    SB = R // Q

    @functools.partial(
        pl.kernel, mesh=mesh,
        out_type=[jax.ShapeDtypeStruct((TOT, C), jnp.float32),
                  jax.ShapeDtypeStruct((NW, 2 * NGR, 16), jnp.float32)],
        scratch_types=[pltpu.VMEM((R * K_NB,), jnp.int32),
                       pltpu.VMEM((Q * K_NB, C), jnp.float32),
                       pltpu.VMEM((Q * K_NB, C), jnp.float32),
                       pltpu.VMEM((Q, C), jnp.float32),
                       pltpu.VMEM((Q, C), jnp.float32),
                       pltpu.VMEM((Q, C), jnp.float32),
                       pltpu.VMEM((Q, C), jnp.float32),
                       pltpu.VMEM((2 * NGR, 16), jnp.float32),
                       pltpu.SemaphoreType.DMA,
                       pltpu.SemaphoreType.DMA,
                       pltpu.SemaphoreType.DMA,
                       pltpu.SemaphoreType.DMA,
                       pltpu.SemaphoreType.DMA,
                       pltpu.SemaphoreType.DMA])
    def sc_gather(idx_hbm, table_hbm, v_hbm, y_hbm, part_hbm,
                  idx_v, rows0, rows1, v0, v1, y0, y1, stat_v,
                  semr0, semr1, semv0, semv1, semy0, semy1):
        wid = lax.axis_index("s") * 2 + lax.axis_index("c")
        base = wid * R
        pltpu.sync_copy(idx_hbm.at[pl.ds(base * K_NB, R * K_NB)], idx_v)
        zeros16 = jnp.zeros((16,), jnp.float32)
        for g in range(2 * NGR):
            stat_v[g, :] = zeros16

        def rows_src(s):
            return table_hbm.at[idx_v.at[pl.ds(s * (Q * K_NB), Q * K_NB)]]

        def v_src(s):
            return v_hbm.at[pl.ds(base + s * Q, Q)]

        def y_dst(s):
            return y_hbm.at[pl.ds(base + s * Q, Q)]

        def start(s, rbuf, vbuf, semr, semv):
            pltpu.async_copy(rows_src(s), rbuf, semr)
            pltpu.async_copy(v_src(s), vbuf, semv)

        def compute(s, rows_v, v_v, y_v, semr, semv, semy):
            pltpu.make_async_copy(rows_src(s), rows_v, semr).wait()
            pltpu.make_async_copy(v_src(s), v_v, semv).wait()

            @pl.when(s >= 2)
            def _():
                pltpu.make_async_copy(y_v, y_dst(s), semy).wait()

            def qbody(q, _):
                r0 = q * K_NB
                for g in range(NGR):
                    gs = zeros16
                    gq = zeros16
                    for cc in range(CHG):
                        sl = pl.ds((g * CHG + cc) * 16, 16)
                        v = rows_v[r0, sl]
                        acc_s = v
                        acc_q = v * v
                        acc_m = v
                        for j in range(1, K_NB):
                            v = rows_v[r0 + j, sl]
                            acc_s = acc_s + v
                            acc_q = acc_q + v * v
                            acc_m = jnp.maximum(acc_m, v)
                        v1 = v_v[q, sl]
                        y_v[q, sl] = acc_m + v1
                        gs = gs + acc_s + float(K_NB) * v1
                        gq = gq + acc_q + v1 * (2.0 * acc_s + float(K_NB) * v1)
                    stat_v[2 * g, :] = stat_v[2 * g, :] + gs
                    stat_v[2 * g + 1, :] = stat_v[2 * g + 1, :] + gq
                return 0

            lax.fori_loop(0, Q, qbody, 0)
            pltpu.async_copy(y_v, y_dst(s), semy)

        start(0, rows0, v0, semr0, semv0)

        def outer(i, _):
            s0 = i * 2
            start(s0 + 1, rows1, v1, semr1, semv1)
            compute(s0, rows0, v0, y0, semr0, semv0, semy0)

            @pl.when(i + 1 < SB // 2)
            def _():
                start(s0 + 2, rows0, v0, semr0, semv0)

            compute(s0 + 1, rows1, v1, y1, semr1, semv1, semy1)
            return 0

        lax.fori_loop(0, SB // 2, outer, 0)
        pltpu.make_async_copy(y0, y_dst(SB - 2), semy0).wait()
        pltpu.make_async_copy(y1, y_dst(SB - 1), semy1).wait()
        pltpu.sync_copy(stat_v, part_hbm.at[wid])

    return sc_gather


def _stats_rows(part_blk, emean_ref, esq_ref):
    ones = jnp.ones((1, 8), jnp.float32)
    srow = lax.dot_general(ones, part_blk, (((1,), (0,)), ((), ())),
                           preferred_element_type=jnp.float32,
                           precision=lax.Precision.HIGHEST)
    meanC = lax.dot_general(srow, emean_ref[...], (((1,), (0,)), ((), ())),
                            preferred_element_type=jnp.float32,
                            precision=lax.Precision.HIGHEST)
    msqC = lax.dot_general(srow, esq_ref[...], (((1,), (0,)), ((), ())),
                           preferred_element_type=jnp.float32,
                           precision=lax.Precision.HIGHEST)
    return meanC, msqC


def _gn_leaky(y, meanC, msqC, gam, bet):
    varC = msqC - meanC * meanC
    rstdC = lax.rsqrt(varC + EPS)
    z = (y - meanC) * (rstdC * gam) + bet
    return jnp.where(z >= 0, z, SLOPE * z)


def _fin1_body(y_ref, part_ref, g_ref, b_ref, em_ref, es_ref, wa_ref, wd_ref,
               u_ref, v_ref):
    meanC, msqC = _stats_rows(part_ref[0], em_ref, es_ref)
    z = _gn_leaky(y_ref[0], meanC, msqC, g_ref[...], b_ref[...])
    u_ref[0] = lax.dot_general(z, wa_ref[...], (((1,), (0,)), ((), ())),
                               preferred_element_type=jnp.float32)
    v_ref[0] = lax.dot_general(z, wd_ref[...], (((1,), (0,)), ((), ())),
                               preferred_element_type=jnp.float32)


def _fin1(y3, part3, g1r, b1r, em, es, waT, wdT):
    B, N, C1 = y3.shape
    C2 = waT.shape[1]
    RT = 512
    return pl.pallas_call(
        _fin1_body,
        grid=(B, N // RT),
        in_specs=[pl.BlockSpec((1, RT, C1), lambda b, t: (b, t, 0)),
                  pl.BlockSpec((1, 8, 128), lambda b, t: (b, 0, 0)),
                  pl.BlockSpec((1, C1), lambda b, t: (0, 0)),
                  pl.BlockSpec((1, C1), lambda b, t: (0, 0)),
                  pl.BlockSpec((128, C1), lambda b, t: (0, 0)),
                  pl.BlockSpec((128, C1), lambda b, t: (0, 0)),
                  pl.BlockSpec((C1, C2), lambda b, t: (0, 0)),
                  pl.BlockSpec((C1, C2), lambda b, t: (0, 0))],
        out_specs=[pl.BlockSpec((1, RT, C2), lambda b, t: (b, t, 0)),
                   pl.BlockSpec((1, RT, C2), lambda b, t: (b, t, 0))],
        out_shape=[jax.ShapeDtypeStruct((B, N, C2), jnp.float32),
                   jax.ShapeDtypeStruct((B, N, C2), jnp.float32)],
    )(y3, part3, g1r, b1r, em, es, waT, wdT)


def _fin2_body(y_ref, part_ref, g_ref, b_ref, em_ref, es_ref, o_ref):
    meanC, msqC = _stats_rows(part_ref[0], em_ref, es_ref)
    z = _gn_leaky(y_ref[0], meanC, msqC, g_ref[...], b_ref[...])
    o_ref[0] = z.T


def _fin2(y3, part3, g2r, b2r, em, es):
    B, N, C2 = y3.shape
    RT = 512
    return pl.pallas_call(
        _fin2_body,
        grid=(B, N // RT),
        in_specs=[pl.BlockSpec((1, RT, C2), lambda b, t: (b, t, 0)),
                  pl.BlockSpec((1, 8, 128), lambda b, t: (b, 0, 0)),
                  pl.BlockSpec((1, C2), lambda b, t: (0, 0)),
                  pl.BlockSpec((1, C2), lambda b, t: (0, 0)),
                  pl.BlockSpec((128, C2), lambda b, t: (0, 0)),
                  pl.BlockSpec((128, C2), lambda b, t: (0, 0))],
        out_specs=pl.BlockSpec((1, C2, RT), lambda b, t: (b, 0, t)),
        out_shape=jax.ShapeDtypeStruct((B, C2, N), jnp.float32),
    )(y3, part3, g2r, b2r, em, es)


def _expand_mats(C, N):
    CPG = C // NGR
    cnt = float(CPG * N * K_NB)
    em = np.zeros((128, C), np.float32)
    es = np.zeros((128, C), np.float32)
    for g in range(NGR):
        for l in range(16):
            em[16 * (2 * g) + l, g * CPG:(g + 1) * CPG] = 1.0 / cnt
            es[16 * (2 * g + 1) + l, g * CPG:(g + 1) * CPG] = 1.0 / cnt
    return jnp.asarray(em), jnp.asarray(es)


def kernel(coor, f, coor_q, f_q, W1, g1, b1, W2, g2, b2):
    B, _, M1 = coor.shape
    N = coor_q.shape[2]
    Cin = f.shape[1]
    C1 = W1.shape[0]
    C2 = W2.shape[0]
    TOT = B * N

    cqT = jnp.transpose(coor_q, (0, 2, 1))
    fT = jnp.transpose(f, (0, 2, 1))
    fqT = jnp.transpose(f_q, (0, 2, 1))
    W1aT = jnp.transpose(W1[:, :Cin])
    W1dT = jnp.transpose(W1[:, Cin:] - W1[:, :Cin])
    W2aT = jnp.transpose(W2[:, :C1])
    W2dT = jnp.transpose(W2[:, C1:] - W2[:, :C1])

    idx1 = _knn(cqT, coor)

    U1 = _gemm(fT, W1aT).reshape(B * M1, C1)
    V1 = _gemm(fqT, W1dT).reshape(TOT, C1)

    sc1 = _make_sc_gather(TOT, C1)
    Y1, P1 = sc1(idx1.reshape(-1), U1, V1)

    idx2 = _knn(cqT, coor_q)

    em1, es1 = _expand_mats(C1, N)
    U2, V2 = _fin1(Y1.reshape(B, N, C1), P1.reshape(B, 8, 128),
                   g1.reshape(1, C1), b1.reshape(1, C1), em1, es1, W2aT, W2dT)

    sc2 = _make_sc_gather(TOT, C2, Q=8)
    Y2, P2 = sc2(idx2.reshape(-1), U2.reshape(TOT, C2), V2.reshape(TOT, C2))

    em2, es2 = _expand_mats(C2, N)
    return _fin2(Y2.reshape(B, N, C2), P2.reshape(B, 8, 128),
                 g2.reshape(1, C2), b2.reshape(1, C2), em2, es2)


# speedup vs baseline: 1.0031x; 1.0031x over previous
import functools

import numpy as np
import jax
import jax.numpy as jnp
from jax import lax
from jax.experimental import pallas as pl
from jax.experimental.pallas import tpu as pltpu
from jax.experimental.pallas import tpu_sc as plsc

K_NB = 16
NGR = 4
EPS = 1e-5
SLOPE = 0.2
NW = 32


def _knn_body(cq_ref, ck_ref, out_ref, *, M):
    b = pl.program_id(0)
    q = cq_ref[0]
    k = ck_ref[0]
    sq_q = q[:, 0:1] * q[:, 0:1] + q[:, 1:2] * q[:, 1:2] + q[:, 2:3] * q[:, 2:3]
    sq_k = k[0:1] * k[0:1] + k[1:2] * k[1:2] + k[2:3] * k[2:3]
    dot = lax.dot_general(q.astype(jnp.bfloat16), k.astype(jnp.bfloat16),
                          (((1,), (0,)), ((), ())),
                          preferred_element_type=jnp.float32)
    d = (sq_q + sq_k) - 2.0 * dot
    iota = lax.broadcasted_iota(jnp.int32, d.shape, 1)
    cols = []
    for _ in range(K_NB):
        m = jnp.min(d, axis=1, keepdims=True)
        eq = d == m
        idxj = jnp.min(jnp.where(eq, iota, M), axis=1, keepdims=True)
        cols.append(idxj)
        d = jnp.where(iota == idxj, jnp.float32(jnp.inf), d)
    out_ref[0] = jnp.concatenate(cols, axis=1) + b * M


def _knn(cqT, ck):
    B, N, _ = cqT.shape
    M = ck.shape[2]
    NT = 512
    return pl.pallas_call(
        functools.partial(_knn_body, M=M),
        grid=(B, N // NT),
        in_specs=[pl.BlockSpec((1, NT, 3), lambda b, t: (b, t, 0)),
                  pl.BlockSpec((1, 3, M), lambda b, t: (b, 0, 0))],
        out_specs=pl.BlockSpec((1, NT, K_NB), lambda b, t: (b, t, 0)),
        out_shape=jax.ShapeDtypeStruct((B, N, K_NB), jnp.int32),
    )(cqT, ck)


def _gemm_body(a_ref, w_ref, o_ref):
    o_ref[0] = lax.dot_general(a_ref[0], w_ref[...], (((1,), (0,)), ((), ())),
                               preferred_element_type=jnp.float32)


def _gemm(a, w):
    B, R, Kd = a.shape
    Co = w.shape[1]
    RT = min(R, 512)
    return pl.pallas_call(
        _gemm_body,
        grid=(B, R // RT),
        in_specs=[pl.BlockSpec((1, RT, Kd), lambda b, t: (b, t, 0)),
                  pl.BlockSpec((Kd, Co), lambda b, t: (0, 0))],
        out_specs=pl.BlockSpec((1, RT, Co), lambda b, t: (b, t, 0)),
        out_shape=jax.ShapeDtypeStruct((B, R, Co), jnp.float32),
    )(a, w)


def _make_sc_gather(TOT, C, Q=4):
    R = TOT // NW
    CH = C // 16
    CHG = CH // NGR
    mesh = plsc.VectorSubcoreMesh(core_axis_name="c", subcore_axis_name="s")

    SB = R // Q

    @functools.partial(
        pl.kernel, mesh=mesh,
        out_type=[jax.ShapeDtypeStruct((TOT, C), jnp.float32),
                  jax.ShapeDtypeStruct((NW, 2 * NGR, 16), jnp.float32)],
        scratch_types=[pltpu.VMEM((R * K_NB,), jnp.int32),
                       pltpu.VMEM((Q * K_NB, C), jnp.float32),
                       pltpu.VMEM((Q * K_NB, C), jnp.float32),
                       pltpu.VMEM((Q, C), jnp.float32),
                       pltpu.VMEM((Q, C), jnp.float32),
                       pltpu.VMEM((Q, C), jnp.float32),
                       pltpu.VMEM((Q, C), jnp.float32),
                       pltpu.VMEM((2 * NGR, 16), jnp.float32),
                       pltpu.SemaphoreType.DMA,
                       pltpu.SemaphoreType.DMA,
                       pltpu.SemaphoreType.DMA,
                       pltpu.SemaphoreType.DMA,
                       pltpu.SemaphoreType.DMA,
                       pltpu.SemaphoreType.DMA])
    def sc_gather(idx_hbm, table_hbm, v_hbm, y_hbm, part_hbm,
                  idx_v, rows0, rows1, v0, v1, y0, y1, stat_v,
                  semr0, semr1, semv0, semv1, semy0, semy1):
        wid = lax.axis_index("s") * 2 + lax.axis_index("c")
        base = wid * R
        pltpu.sync_copy(idx_hbm.at[pl.ds(base * K_NB, R * K_NB)], idx_v)
        zeros16 = jnp.zeros((16,), jnp.float32)
        for g in range(2 * NGR):
            stat_v[g, :] = zeros16

        def rows_src(s):
            return table_hbm.at[idx_v.at[pl.ds(s * (Q * K_NB), Q * K_NB)]]

        def v_src(s):
            return v_hbm.at[pl.ds(base + s * Q, Q)]

        def y_dst(s):
            return y_hbm.at[pl.ds(base + s * Q, Q)]

        def start(s, rbuf, vbuf, semr, semv):
            pltpu.async_copy(rows_src(s), rbuf, semr)
            pltpu.async_copy(v_src(s), vbuf, semv)

        def compute(s, rows_v, v_v, y_v, semr, semv, semy):
            pltpu.make_async_copy(rows_src(s), rows_v, semr).wait()
            pltpu.make_async_copy(v_src(s), v_v, semv).wait()

            @pl.when(s >= 2)
            def _():
                pltpu.make_async_copy(y_v, y_dst(s), semy).wait()

            def qbody(q, _):
                r0 = q * K_NB
                for g in range(NGR):
                    gs = zeros16
                    gq = zeros16
                    for cc in range(CHG):
                        sl = pl.ds((g * CHG + cc) * 16, 16)
                        v = rows_v[r0, sl]
                        acc_s = v
                        acc_q = v * v
                        acc_m = v
                        for j in range(1, K_NB):
                            v = rows_v[r0 + j, sl]
                            acc_s = acc_s + v
                            acc_q = acc_q + v * v
                            acc_m = jnp.maximum(acc_m, v)
                        v1 = v_v[q, sl]
                        y_v[q, sl] = acc_m + v1
                        gs = gs + acc_s + float(K_NB) * v1
                        gq = gq + acc_q + v1 * (2.0 * acc_s + float(K_NB) * v1)
                    stat_v[2 * g, :] = stat_v[2 * g, :] + gs
                    stat_v[2 * g + 1, :] = stat_v[2 * g + 1, :] + gq
                return 0

            lax.fori_loop(0, Q, qbody, 0)
            pltpu.async_copy(y_v, y_dst(s), semy)

        start(0, rows0, v0, semr0, semv0)

        def outer(i, _):
            s0 = i * 2
            start(s0 + 1, rows1, v1, semr1, semv1)
            compute(s0, rows0, v0, y0, semr0, semv0, semy0)

            @pl.when(i + 1 < SB // 2)
            def _():
                start(s0 + 2, rows0, v0, semr0, semv0)

            compute(s0 + 1, rows1, v1, y1, semr1, semv1, semy1)
            return 0

        lax.fori_loop(0, SB // 2, outer, 0)
        pltpu.make_async_copy(y0, y_dst(SB - 2), semy0).wait()
        pltpu.make_async_copy(y1, y_dst(SB - 1), semy1).wait()
        pltpu.sync_copy(stat_v, part_hbm.at[wid])

    return sc_gather


def _stats_rows(part_blk, emean_ref, esq_ref):
    ones = jnp.ones((1, 8), jnp.float32)
    srow = lax.dot_general(ones, part_blk, (((1,), (0,)), ((), ())),
                           preferred_element_type=jnp.float32,
                           precision=lax.Precision.HIGHEST)
    meanC = lax.dot_general(srow, emean_ref[...], (((1,), (0,)), ((), ())),
                            preferred_element_type=jnp.float32,
                            precision=lax.Precision.HIGHEST)
    msqC = lax.dot_general(srow, esq_ref[...], (((1,), (0,)), ((), ())),
                           preferred_element_type=jnp.float32,
                           precision=lax.Precision.HIGHEST)
    return meanC, msqC


def _gn_leaky(y, meanC, msqC, gam, bet):
    varC = msqC - meanC * meanC
    rstdC = lax.rsqrt(varC + EPS)
    z = (y - meanC) * (rstdC * gam) + bet
    return jnp.where(z >= 0, z, SLOPE * z)


def _fin1_body(y_ref, part_ref, g_ref, b_ref, em_ref, es_ref, wa_ref, wd_ref,
               u_ref, v_ref):
    meanC, msqC = _stats_rows(part_ref[0], em_ref, es_ref)
    z = _gn_leaky(y_ref[0], meanC, msqC, g_ref[...], b_ref[...])
    u_ref[0] = lax.dot_general(z, wa_ref[...], (((1,), (0,)), ((), ())),
                               preferred_element_type=jnp.float32)
    v_ref[0] = lax.dot_general(z, wd_ref[...], (((1,), (0,)), ((), ())),
                               preferred_element_type=jnp.float32)


def _fin1(y3, part3, g1r, b1r, em, es, waT, wdT):
    B, N, C1 = y3.shape
    C2 = waT.shape[1]
    RT = 256
    return pl.pallas_call(
        _fin1_body,
        grid=(B, N // RT),
        in_specs=[pl.BlockSpec((1, RT, C1), lambda b, t: (b, t, 0)),
                  pl.BlockSpec((1, 8, 128), lambda b, t: (b, 0, 0)),
                  pl.BlockSpec((1, C1), lambda b, t: (0, 0)),
                  pl.BlockSpec((1, C1), lambda b, t: (0, 0)),
                  pl.BlockSpec((128, C1), lambda b, t: (0, 0)),
                  pl.BlockSpec((128, C1), lambda b, t: (0, 0)),
                  pl.BlockSpec((C1, C2), lambda b, t: (0, 0)),
                  pl.BlockSpec((C1, C2), lambda b, t: (0, 0))],
        out_specs=[pl.BlockSpec((1, RT, C2), lambda b, t: (b, t, 0)),
                   pl.BlockSpec((1, RT, C2), lambda b, t: (b, t, 0))],
        out_shape=[jax.ShapeDtypeStruct((B, N, C2), jnp.float32),
                   jax.ShapeDtypeStruct((B, N, C2), jnp.float32)],
    )(y3, part3, g1r, b1r, em, es, waT, wdT)


def _fin2_body(y_ref, part_ref, g_ref, b_ref, em_ref, es_ref, o_ref):
    meanC, msqC = _stats_rows(part_ref[0], em_ref, es_ref)
    z = _gn_leaky(y_ref[0], meanC, msqC, g_ref[...], b_ref[...])
    o_ref[0] = z.T


def _fin2(y3, part3, g2r, b2r, em, es):
    B, N, C2 = y3.shape
    RT = 256
    return pl.pallas_call(
        _fin2_body,
        grid=(B, N // RT),
        in_specs=[pl.BlockSpec((1, RT, C2), lambda b, t: (b, t, 0)),
                  pl.BlockSpec((1, 8, 128), lambda b, t: (b, 0, 0)),
                  pl.BlockSpec((1, C2), lambda b, t: (0, 0)),
                  pl.BlockSpec((1, C2), lambda b, t: (0, 0)),
                  pl.BlockSpec((128, C2), lambda b, t: (0, 0)),
                  pl.BlockSpec((128, C2), lambda b, t: (0, 0))],
        out_specs=pl.BlockSpec((1, C2, RT), lambda b, t: (b, 0, t)),
        out_shape=jax.ShapeDtypeStruct((B, C2, N), jnp.float32),
    )(y3, part3, g2r, b2r, em, es)


def _expand_mats(C, N):
    CPG = C // NGR
    cnt = float(CPG * N * K_NB)
    em = np.zeros((128, C), np.float32)
    es = np.zeros((128, C), np.float32)
    for g in range(NGR):
        for l in range(16):
            em[16 * (2 * g) + l, g * CPG:(g + 1) * CPG] = 1.0 / cnt
            es[16 * (2 * g + 1) + l, g * CPG:(g + 1) * CPG] = 1.0 / cnt
    return jnp.asarray(em), jnp.asarray(es)


def kernel(coor, f, coor_q, f_q, W1, g1, b1, W2, g2, b2):
    B, _, M1 = coor.shape
    N = coor_q.shape[2]
    Cin = f.shape[1]
    C1 = W1.shape[0]
    C2 = W2.shape[0]
    TOT = B * N

    cqT = jnp.transpose(coor_q, (0, 2, 1))
    fT = jnp.transpose(f, (0, 2, 1))
    fqT = jnp.transpose(f_q, (0, 2, 1))
    W1aT = jnp.transpose(W1[:, :Cin])
    W1dT = jnp.transpose(W1[:, Cin:] - W1[:, :Cin])
    W2aT = jnp.transpose(W2[:, :C1])
    W2dT = jnp.transpose(W2[:, C1:] - W2[:, :C1])

    idx1 = _knn(cqT, coor)

    U1 = _gemm(fT, W1aT).reshape(B * M1, C1)
    V1 = _gemm(fqT, W1dT).reshape(TOT, C1)

    sc1 = _make_sc_gather(TOT, C1)
    Y1, P1 = sc1(idx1.reshape(-1), U1, V1)

    idx2 = _knn(cqT, coor_q)

    em1, es1 = _expand_mats(C1, N)
    U2, V2 = _fin1(Y1.reshape(B, N, C1), P1.reshape(B, 8, 128),
                   g1.reshape(1, C1), b1.reshape(1, C1), em1, es1, W2aT, W2dT)

    sc2 = _make_sc_gather(TOT, C2, Q=8)
    Y2, P2 = sc2(idx2.reshape(-1), U2.reshape(TOT, C2), V2.reshape(TOT, C2))

    em2, es2 = _expand_mats(C2, N)
    return _fin2(Y2.reshape(B, N, C2), P2.reshape(B, 8, 128),
                 g2.reshape(1, C2), b2.reshape(1, C2), em2, es2)


# speedup vs baseline: 1.0039x; 1.0008x over previous
import functools

import numpy as np
import jax
import jax.numpy as jnp
from jax import lax
from jax.experimental import pallas as pl
from jax.experimental.pallas import tpu as pltpu
from jax.experimental.pallas import tpu_sc as plsc

K_NB = 16
NGR = 4
EPS = 1e-5
SLOPE = 0.2
NW = 32


def _knn_body(cq_ref, ck_ref, out_ref, *, M):
    b = pl.program_id(0)
    q = cq_ref[0]
    k = ck_ref[0]
    sq_q = q[:, 0:1] * q[:, 0:1] + q[:, 1:2] * q[:, 1:2] + q[:, 2:3] * q[:, 2:3]
    sq_k = k[0:1] * k[0:1] + k[1:2] * k[1:2] + k[2:3] * k[2:3]
    dot = lax.dot_general(q.astype(jnp.bfloat16), k.astype(jnp.bfloat16),
                          (((1,), (0,)), ((), ())),
                          preferred_element_type=jnp.float32)
    d = (sq_q + sq_k) - 2.0 * dot
    iota = lax.broadcasted_iota(jnp.int32, d.shape, 1)
    cols = []
    for _ in range(K_NB):
        m = jnp.min(d, axis=1, keepdims=True)
        eq = d == m
        idxj = jnp.min(jnp.where(eq, iota, M), axis=1, keepdims=True)
        cols.append(idxj)
        d = jnp.where(iota == idxj, jnp.float32(jnp.inf), d)
    out_ref[0] = jnp.concatenate(cols, axis=1) + b * M


def _knn(cqT, ck):
    B, N, _ = cqT.shape
    M = ck.shape[2]
    NT = 512
    return pl.pallas_call(
        functools.partial(_knn_body, M=M),
        grid=(B, N // NT),
        in_specs=[pl.BlockSpec((1, NT, 3), lambda b, t: (b, t, 0)),
                  pl.BlockSpec((1, 3, M), lambda b, t: (b, 0, 0))],
        out_specs=pl.BlockSpec((1, NT, K_NB), lambda b, t: (b, t, 0)),
        out_shape=jax.ShapeDtypeStruct((B, N, K_NB), jnp.int32),
    )(cqT, ck)


def _gemm_body(a_ref, w_ref, o_ref):
    o_ref[0] = lax.dot_general(a_ref[0], w_ref[...], (((1,), (0,)), ((), ())),
                               preferred_element_type=jnp.float32)


def _gemm(a, w):
    B, R, Kd = a.shape
    Co = w.shape[1]
    RT = min(R, 512)
    return pl.pallas_call(
        _gemm_body,
        grid=(B, R // RT),
        in_specs=[pl.BlockSpec((1, RT, Kd), lambda b, t: (b, t, 0)),
                  pl.BlockSpec((Kd, Co), lambda b, t: (0, 0))],
        out_specs=pl.BlockSpec((1, RT, Co), lambda b, t: (b, t, 0)),
        out_shape=jax.ShapeDtypeStruct((B, R, Co), jnp.float32),
    )(a, w)


def _make_sc_gather(TOT, C, Q=4):
    R = TOT // NW
    CH = C // 16
    CHG = CH // NGR
    mesh = plsc.VectorSubcoreMesh(core_axis_name="c", subcore_axis_name="s")

    SB = R // Q

    @functools.partial(
        pl.kernel, mesh=mesh,
        out_type=[jax.ShapeDtypeStruct((TOT, C), jnp.float32),
                  jax.ShapeDtypeStruct((NW, 2 * NGR, 16), jnp.float32)],
        scratch_types=[pltpu.VMEM((R * K_NB,), jnp.int32),
                       pltpu.VMEM((Q * K_NB, C), jnp.float32),
                       pltpu.VMEM((Q * K_NB, C), jnp.float32),
                       pltpu.VMEM((Q, C), jnp.float32),
                       pltpu.VMEM((Q, C), jnp.float32),
                       pltpu.VMEM((Q, C), jnp.float32),
                       pltpu.VMEM((Q, C), jnp.float32),
                       pltpu.VMEM((2 * NGR, 16), jnp.float32),
                       pltpu.SemaphoreType.DMA,
                       pltpu.SemaphoreType.DMA,
                       pltpu.SemaphoreType.DMA,
                       pltpu.SemaphoreType.DMA,
                       pltpu.SemaphoreType.DMA,
                       pltpu.SemaphoreType.DMA])
    def sc_gather(idx_hbm, table_hbm, v_hbm, y_hbm, part_hbm,
                  idx_v, rows0, rows1, v0, v1, y0, y1, stat_v,
                  semr0, semr1, semv0, semv1, semy0, semy1):
        wid = lax.axis_index("s") * 2 + lax.axis_index("c")
        base = wid * R
        pltpu.sync_copy(idx_hbm.at[pl.ds(base * K_NB, R * K_NB)], idx_v)
        zeros16 = jnp.zeros((16,), jnp.float32)
        for g in range(2 * NGR):
            stat_v[g, :] = zeros16

        def rows_src(s):
            return table_hbm.at[idx_v.at[pl.ds(s * (Q * K_NB), Q * K_NB)]]

        def v_src(s):
            return v_hbm.at[pl.ds(base + s * Q, Q)]

        def y_dst(s):
            return y_hbm.at[pl.ds(base + s * Q, Q)]

        def start(s, rbuf, vbuf, semr, semv):
            pltpu.async_copy(rows_src(s), rbuf, semr)
            pltpu.async_copy(v_src(s), vbuf, semv)

        def compute(s, rows_v, v_v, y_v, semr, semv, semy):
            pltpu.make_async_copy(rows_src(s), rows_v, semr).wait()
            pltpu.make_async_copy(v_src(s), v_v, semv).wait()

            @pl.when(s >= 2)
            def _():
                pltpu.make_async_copy(y_v, y_dst(s), semy).wait()

            def qbody(q, _):
                r0 = q * K_NB
                for g in range(NGR):
                    gs = zeros16
                    gq = zeros16
                    for cc in range(CHG):
                        sl = pl.ds((g * CHG + cc) * 16, 16)
                        v = rows_v[r0, sl]
                        acc_s = v
                        acc_q = v * v
                        acc_m = v
                        for j in range(1, K_NB):
                            v = rows_v[r0 + j, sl]
                            acc_s = acc_s + v
                            acc_q = acc_q + v * v
                            acc_m = jnp.maximum(acc_m, v)
                        v1 = v_v[q, sl]
                        y_v[q, sl] = acc_m + v1
                        gs = gs + acc_s + float(K_NB) * v1
                        gq = gq + acc_q + v1 * (2.0 * acc_s + float(K_NB) * v1)
                    stat_v[2 * g, :] = stat_v[2 * g, :] + gs
                    stat_v[2 * g + 1, :] = stat_v[2 * g + 1, :] + gq
                return 0

            lax.fori_loop(0, Q, qbody, 0)
            pltpu.async_copy(y_v, y_dst(s), semy)

        start(0, rows0, v0, semr0, semv0)

        def outer(i, _):
            s0 = i * 2
            start(s0 + 1, rows1, v1, semr1, semv1)
            compute(s0, rows0, v0, y0, semr0, semv0, semy0)

            @pl.when(i + 1 < SB // 2)
            def _():
                start(s0 + 2, rows0, v0, semr0, semv0)

            compute(s0 + 1, rows1, v1, y1, semr1, semv1, semy1)
            return 0

        lax.fori_loop(0, SB // 2, outer, 0)
        pltpu.make_async_copy(y0, y_dst(SB - 2), semy0).wait()
        pltpu.make_async_copy(y1, y_dst(SB - 1), semy1).wait()
        pltpu.sync_copy(stat_v, part_hbm.at[wid])

    return sc_gather


def _stats_rows(part_blk, emean_ref, esq_ref):
    ones = jnp.ones((1, 8), jnp.float32)
    srow = lax.dot_general(ones, part_blk, (((1,), (0,)), ((), ())),
                           preferred_element_type=jnp.float32,
                           precision=lax.Precision.HIGHEST)
    meanC = lax.dot_general(srow, emean_ref[...], (((1,), (0,)), ((), ())),
                            preferred_element_type=jnp.float32,
                            precision=lax.Precision.HIGHEST)
    msqC = lax.dot_general(srow, esq_ref[...], (((1,), (0,)), ((), ())),
                           preferred_element_type=jnp.float32,
                           precision=lax.Precision.HIGHEST)
    return meanC, msqC


def _gn_leaky(y, meanC, msqC, gam, bet):
    varC = msqC - meanC * meanC
    rstdC = lax.rsqrt(varC + EPS)
    z = (y - meanC) * (rstdC * gam) + bet
    return jnp.where(z >= 0, z, SLOPE * z)


def _fin1_body(y_ref, part_ref, g_ref, b_ref, em_ref, es_ref, w_ref,
               u_ref, v_ref):
    meanC, msqC = _stats_rows(part_ref[0], em_ref, es_ref)
    z = _gn_leaky(y_ref[0], meanC, msqC, g_ref[...], b_ref[...])
    uv = lax.dot_general(z, w_ref[...], (((1,), (0,)), ((), ())),
                         preferred_element_type=jnp.float32)
    C2 = uv.shape[1] // 2
    u_ref[0] = uv[:, :C2]
    v_ref[0] = uv[:, C2:]


def _fin1(y3, part3, g1r, b1r, em, es, wcat):
    B, N, C1 = y3.shape
    C2 = wcat.shape[1] // 2
    RT = 256
    return pl.pallas_call(
        _fin1_body,
        grid=(B, N // RT),
        in_specs=[pl.BlockSpec((1, RT, C1), lambda b, t: (b, t, 0)),
                  pl.BlockSpec((1, 8, 128), lambda b, t: (b, 0, 0)),
                  pl.BlockSpec((1, C1), lambda b, t: (0, 0)),
                  pl.BlockSpec((1, C1), lambda b, t: (0, 0)),
                  pl.BlockSpec((128, C1), lambda b, t: (0, 0)),
                  pl.BlockSpec((128, C1), lambda b, t: (0, 0)),
                  pl.BlockSpec((C1, 2 * C2), lambda b, t: (0, 0))],
        out_specs=[pl.BlockSpec((1, RT, C2), lambda b, t: (b, t, 0)),
                   pl.BlockSpec((1, RT, C2), lambda b, t: (b, t, 0))],
        out_shape=[jax.ShapeDtypeStruct((B, N, C2), jnp.float32),
                   jax.ShapeDtypeStruct((B, N, C2), jnp.float32)],
    )(y3, part3, g1r, b1r, em, es, wcat)


def _fin2_body(y_ref, part_ref, g_ref, b_ref, em_ref, es_ref, o_ref):
    meanC, msqC = _stats_rows(part_ref[0], em_ref, es_ref)
    z = _gn_leaky(y_ref[0], meanC, msqC, g_ref[...], b_ref[...])
    o_ref[0] = z.T


def _fin2(y3, part3, g2r, b2r, em, es):
    B, N, C2 = y3.shape
    RT = 256
    return pl.pallas_call(
        _fin2_body,
        grid=(B, N // RT),
        in_specs=[pl.BlockSpec((1, RT, C2), lambda b, t: (b, t, 0)),
                  pl.BlockSpec((1, 8, 128), lambda b, t: (b, 0, 0)),
                  pl.BlockSpec((1, C2), lambda b, t: (0, 0)),
                  pl.BlockSpec((1, C2), lambda b, t: (0, 0)),
                  pl.BlockSpec((128, C2), lambda b, t: (0, 0)),
                  pl.BlockSpec((128, C2), lambda b, t: (0, 0))],
        out_specs=pl.BlockSpec((1, C2, RT), lambda b, t: (b, 0, t)),
        out_shape=jax.ShapeDtypeStruct((B, C2, N), jnp.float32),
    )(y3, part3, g2r, b2r, em, es)


def _expand_mats(C, N):
    CPG = C // NGR
    cnt = float(CPG * N * K_NB)
    em = np.zeros((128, C), np.float32)
    es = np.zeros((128, C), np.float32)
    for g in range(NGR):
        for l in range(16):
            em[16 * (2 * g) + l, g * CPG:(g + 1) * CPG] = 1.0 / cnt
            es[16 * (2 * g + 1) + l, g * CPG:(g + 1) * CPG] = 1.0 / cnt
    return jnp.asarray(em), jnp.asarray(es)


def kernel(coor, f, coor_q, f_q, W1, g1, b1, W2, g2, b2):
    B, _, M1 = coor.shape
    N = coor_q.shape[2]
    Cin = f.shape[1]
    C1 = W1.shape[0]
    C2 = W2.shape[0]
    TOT = B * N

    cqT = jnp.transpose(coor_q, (0, 2, 1))
    fT = jnp.transpose(f, (0, 2, 1))
    fqT = jnp.transpose(f_q, (0, 2, 1))
    W1aT = jnp.transpose(W1[:, :Cin])
    W1dT = jnp.transpose(W1[:, Cin:] - W1[:, :Cin])
    W2aT = jnp.transpose(W2[:, :C1])
    W2dT = jnp.transpose(W2[:, C1:] - W2[:, :C1])

    idx1 = _knn(cqT, coor)

    U1 = _gemm(fT, W1aT).reshape(B * M1, C1)
    V1 = _gemm(fqT, W1dT).reshape(TOT, C1)

    sc1 = _make_sc_gather(TOT, C1)
    Y1, P1 = sc1(idx1.reshape(-1), U1, V1)

    idx2 = _knn(cqT, coor_q)

    em1, es1 = _expand_mats(C1, N)
    U2, V2 = _fin1(Y1.reshape(B, N, C1), P1.reshape(B, 8, 128),
                   g1.reshape(1, C1), b1.reshape(1, C1), em1, es1,
                   jnp.concatenate([W2aT, W2dT], axis=1))

    sc2 = _make_sc_gather(TOT, C2, Q=8)
    Y2, P2 = sc2(idx2.reshape(-1), U2.reshape(TOT, C2), V2.reshape(TOT, C2))

    em2, es2 = _expand_mats(C2, N)
    return _fin2(Y2.reshape(B, N, C2), P2.reshape(B, 8, 128),
                 g2.reshape(1, C2), b2.reshape(1, C2), em2, es2)
